# baseline (speedup 1.0000x reference)
import jax
import jax.numpy as jnp
from jax.experimental import pallas as pl

_BR = 8
_CW = 4096
_EW = 1024
_PASSES = 6


def _pattern_to_float(p):
    pi = jax.lax.bitcast_convert_type(p, jnp.int32)
    s = jnp.where(pi < 0, pi ^ jnp.int32(-2147483648), ~pi)
    return jax.lax.bitcast_convert_type(s, jnp.float32)


def _count_ge(x_ref, tf, v, cw):
    nfull = v // cw
    tail0 = nfull * cw

    def chunk_body(j, acc):
        c = x_ref[:, pl.ds(j * cw, cw)]
        return acc + jnp.where(c >= tf, jnp.float32(1.0), jnp.float32(0.0))

    acc = jnp.zeros((_BR, cw), jnp.float32)
    acc = jax.lax.fori_loop(0, nfull, chunk_body, acc, unroll=8)
    c = jnp.sum(acc, axis=-1, keepdims=True)
    if tail0 < v:
        t = x_ref[:, tail0:v]
        c = c + jnp.sum(jnp.where(t >= tf, jnp.float32(1.0), jnp.float32(0.0)),
                        axis=-1, keepdims=True)
    return c


def _rotl(x, r):
    return (x << jnp.uint32(r)) | (x >> jnp.uint32(32 - r))


def _threefry_bits(k0, k1, lo):
    ks2 = k0 ^ k1 ^ jnp.uint32(0x1BD11BDA)
    rot = ((13, 15, 26, 6), (17, 29, 16, 24))
    ks = (k1, ks2, k0)
    x0 = jnp.zeros_like(lo) + k0
    x1 = lo + k1
    for i in range(5):
        for r in rot[i % 2]:
            x0 = x0 + x1
            x1 = _rotl(x1, r) ^ x0
        x0 = x0 + ks[i % 3]
        x1 = x1 + ks[(i + 1) % 3] + jnp.uint32(i + 1)
    return x0 ^ x1


def _gumbel_from_bits(bits):
    tiny = jnp.float32(1.1754943508222875e-38)
    fb = (bits >> jnp.uint32(9)) | jnp.uint32(0x3F800000)
    fl = jax.lax.bitcast_convert_type(fb, jnp.float32) - jnp.float32(1.0)
    u = jnp.maximum(tiny, fl + tiny)
    return -jnp.log(-jnp.log(u))


def _sample_kernel(x_ref, tk_ref, kr_ref, out_ref):
    kf = tk_ref[0, 0]
    v = x_ref.shape[1]

    def bit_body(i, t):
        bit = jnp.uint32(1) << (jnp.uint32(31) - i.astype(jnp.uint32))
        t_try = t | bit
        tf = _pattern_to_float(t_try)
        c = _count_ge(x_ref, tf, v, _CW)
        return jnp.where(c >= kf, t_try, t)

    t0 = jnp.zeros((_BR, 1), jnp.uint32)
    thr = jax.lax.fori_loop(0, 32, bit_body, t0)
    thr_f = _pattern_to_float(thr)

    ncf = v // _EW
    tail_w = v - ncf * _EW
    lane = jax.lax.broadcasted_iota(jnp.int32, (_BR, _EW), 1)
    rowg = (pl.program_id(0) * _BR
            + jax.lax.broadcasted_iota(jnp.int32, (_BR, _EW), 0))
    k0 = kr_ref[0, 0]
    k1 = kr_ref[0, 1]

    best = jnp.full((_BR, _EW), -jnp.inf, jnp.float32)
    bestcol = jnp.full((_BR, _EW), jnp.int32(2**31 - 1), jnp.int32)
    pvj = jnp.full((_BR, _EW), -1, jnp.int32)

    tail = x_ref[:, ncf * _EW:v]
    tail = jnp.concatenate(
        [tail, jnp.full((_BR, _EW - tail_w), -jnp.inf, jnp.float32)], axis=1)

    for _ in range(_PASSES):
        capv = jnp.zeros((_BR, _EW), jnp.float32)
        capj = jnp.full((_BR, _EW), -1, jnp.int32)

        def chunk_body(j, st):
            capv, capj = st
            c = x_ref[:, pl.ds(j * _EW, _EW)]
            elig = (c >= thr_f) & (j > pvj) & (capj < 0)
            capv = jnp.where(elig, c, capv)
            capj = jnp.where(elig, j, capj)
            return capv, capj

        capv, capj = jax.lax.fori_loop(0, ncf, chunk_body, (capv, capj),
                                       unroll=16)
        elig = (tail >= thr_f) & (ncf > pvj) & (capj < 0)
        capv = jnp.where(elig, tail, capv)
        capj = jnp.where(elig, ncf, capj)
        got = capj >= 0
        pvj = jnp.where(got, capj, jnp.int32(2**31 - 1))

        col = capj * _EW + lane
        flat = jnp.where(got, rowg * v + col, 0).astype(jnp.uint32)
        bits = _threefry_bits(k0, k1, flat)
        score = jnp.where(got, capv + _gumbel_from_bits(bits),
                          jnp.float32(-jnp.inf))
        better = (score > best) | ((score == best) & (col < bestcol))
        best = jnp.where(better, score, best)
        bestcol = jnp.where(better & got, col, bestcol)

    m = jnp.max(best, axis=-1, keepdims=True)
    token = jnp.min(jnp.where(best == m, bestcol, jnp.int32(2**31 - 1)),
                    axis=-1)
    out_ref[0, 0, :] = token


def _build_call(R, V):
    return pl.pallas_call(
        _sample_kernel,
        grid=(R // _BR,),
        in_specs=[
            pl.BlockSpec((_BR, V), lambda i: (i, 0)),
            pl.BlockSpec((1, 1), lambda i: (0, 0)),
            pl.BlockSpec((1, 2), lambda i: (0, 0)),
        ],
        out_specs=pl.BlockSpec((1, 1, _BR), lambda i: (i, 0, 0)),
        out_shape=jax.ShapeDtypeStruct((R // _BR, 1, _BR), jnp.int32),
    )


def kernel(logits, top_k):
    logits = logits.astype(jnp.float32)
    R, V = logits.shape
    sample_key = jax.random.fold_in(jax.random.key(0), 1)
    kr = jax.random.key_data(sample_key).astype(jnp.uint32).reshape(1, 2)
    tk = jnp.asarray(top_k, jnp.float32).reshape(1, 1)
    out = _build_call(R, V)(logits, tk, kr)
    return out.reshape(R)


# speedup vs baseline: 1.1692x; 1.1692x over previous
import jax
import jax.numpy as jnp
from jax.experimental import pallas as pl

_BR = 8
_CW = 2048
_EW = 1024
_PASSES = 6


def _pattern_to_float(p):
    pi = jax.lax.bitcast_convert_type(p, jnp.int32)
    s = jnp.where(pi < 0, pi ^ jnp.int32(-2147483648), ~pi)
    return jax.lax.bitcast_convert_type(s, jnp.float32)


def _count_ge(x_ref, tf, v, cw):
    nfull = v // cw
    tail0 = nfull * cw

    def chunk_body(j, acc):
        c = x_ref[:, pl.ds(j * cw, cw)]
        return acc + jnp.where(c >= tf, jnp.float32(1.0), jnp.float32(0.0))

    acc = jnp.zeros((_BR, cw), jnp.float32)
    acc = jax.lax.fori_loop(0, nfull, chunk_body, acc, unroll=16)
    c = jnp.sum(acc, axis=-1, keepdims=True)
    if tail0 < v:
        t = x_ref[:, tail0:v]
        c = c + jnp.sum(jnp.where(t >= tf, jnp.float32(1.0), jnp.float32(0.0)),
                        axis=-1, keepdims=True)
    return c


def _rotl(x, r):
    return (x << jnp.uint32(r)) | (x >> jnp.uint32(32 - r))


def _threefry_bits(k0, k1, lo):
    ks2 = k0 ^ k1 ^ jnp.uint32(0x1BD11BDA)
    rot = ((13, 15, 26, 6), (17, 29, 16, 24))
    ks = (k1, ks2, k0)
    x0 = jnp.zeros_like(lo) + k0
    x1 = lo + k1
    for i in range(5):
        for r in rot[i % 2]:
            x0 = x0 + x1
            x1 = _rotl(x1, r) ^ x0
        x0 = x0 + ks[i % 3]
        x1 = x1 + ks[(i + 1) % 3] + jnp.uint32(i + 1)
    return x0 ^ x1


def _gumbel_from_bits(bits):
    tiny = jnp.float32(1.1754943508222875e-38)
    fb = (bits >> jnp.uint32(9)) | jnp.uint32(0x3F800000)
    fl = jax.lax.bitcast_convert_type(fb, jnp.float32) - jnp.float32(1.0)
    u = jnp.maximum(tiny, fl + tiny)
    return -jnp.log(-jnp.log(u))


def _float_to_pattern(x):
    s = jax.lax.bitcast_convert_type(x, jnp.int32)
    m = jax.lax.shift_right_arithmetic(s, 31)
    return jax.lax.bitcast_convert_type(s ^ (m | jnp.int32(-2147483648)),
                                        jnp.uint32)


def _slot_max(x_ref, v, cw):
    nfull = v // cw
    tail_w = v - nfull * cw
    t = x_ref[:, nfull * cw:v]
    acc = jnp.concatenate(
        [t, jnp.full((_BR, cw - tail_w), -jnp.inf, jnp.float32)], axis=1)

    def chunk_body(j, acc):
        return jnp.maximum(acc, x_ref[:, pl.ds(j * cw, cw)])

    return jax.lax.fori_loop(0, nfull, chunk_body, acc, unroll=16)


def _sample_kernel(x_ref, tk_ref, kr_ref, out_ref):
    kf = tk_ref[0, 0]
    v = x_ref.shape[1]

    mx = _slot_max(x_ref, v, _CW)
    lo = _float_to_pattern(jnp.min(mx, axis=-1, keepdims=True))
    hi = _float_to_pattern(jnp.max(mx, axis=-1, keepdims=True)) + jnp.uint32(1)

    def cond(st):
        lo, hi = st
        return jnp.any((hi - lo) > jnp.uint32(1))

    def body(st):
        lo, hi = st
        mid = lo + ((hi - lo) >> jnp.uint32(1))
        c = _count_ge(x_ref, _pattern_to_float(mid), v, _CW)
        take = c >= kf
        return jnp.where(take, mid, lo), jnp.where(take, hi, mid)

    lo, hi = jax.lax.while_loop(cond, body, (lo, hi))
    thr_f = _pattern_to_float(lo)

    ncf = v // _EW
    tail_w = v - ncf * _EW
    lane = jax.lax.broadcasted_iota(jnp.int32, (_BR, _EW), 1)
    rowg = (pl.program_id(0) * _BR
            + jax.lax.broadcasted_iota(jnp.int32, (_BR, _EW), 0))
    k0 = kr_ref[0, 0]
    k1 = kr_ref[0, 1]

    best = jnp.full((_BR, _EW), -jnp.inf, jnp.float32)
    bestcol = jnp.full((_BR, _EW), jnp.int32(2**31 - 1), jnp.int32)
    pvj = jnp.full((_BR, _EW), -1, jnp.int32)

    tail = x_ref[:, ncf * _EW:v]
    tail = jnp.concatenate(
        [tail, jnp.full((_BR, _EW - tail_w), -jnp.inf, jnp.float32)], axis=1)

    for _ in range(_PASSES):
        capv = jnp.zeros((_BR, _EW), jnp.float32)
        capj = jnp.full((_BR, _EW), -1, jnp.int32)

        def chunk_body(j, st):
            capv, capj = st
            c = x_ref[:, pl.ds(j * _EW, _EW)]
            elig = (c >= thr_f) & (j > pvj) & (capj < 0)
            capv = jnp.where(elig, c, capv)
            capj = jnp.where(elig, j, capj)
            return capv, capj

        capv, capj = jax.lax.fori_loop(0, ncf, chunk_body, (capv, capj),
                                       unroll=16)
        elig = (tail >= thr_f) & (ncf > pvj) & (capj < 0)
        capv = jnp.where(elig, tail, capv)
        capj = jnp.where(elig, ncf, capj)
        got = capj >= 0
        pvj = jnp.where(got, capj, jnp.int32(2**31 - 1))

        col = capj * _EW + lane
        flat = jnp.where(got, rowg * v + col, 0).astype(jnp.uint32)
        bits = _threefry_bits(k0, k1, flat)
        score = jnp.where(got, capv + _gumbel_from_bits(bits),
                          jnp.float32(-jnp.inf))
        better = (score > best) | ((score == best) & (col < bestcol))
        best = jnp.where(better, score, best)
        bestcol = jnp.where(better & got, col, bestcol)

    m = jnp.max(best, axis=-1, keepdims=True)
    token = jnp.min(jnp.where(best == m, bestcol, jnp.int32(2**31 - 1)),
                    axis=-1)
    out_ref[0, 0, :] = token


def _build_call(R, V):
    return pl.pallas_call(
        _sample_kernel,
        grid=(R // _BR,),
        in_specs=[
            pl.BlockSpec((_BR, V), lambda i: (i, 0)),
            pl.BlockSpec((1, 1), lambda i: (0, 0)),
            pl.BlockSpec((1, 2), lambda i: (0, 0)),
        ],
        out_specs=pl.BlockSpec((1, 1, _BR), lambda i: (i, 0, 0)),
        out_shape=jax.ShapeDtypeStruct((R // _BR, 1, _BR), jnp.int32),
    )


def kernel(logits, top_k):
    logits = logits.astype(jnp.float32)
    R, V = logits.shape
    sample_key = jax.random.fold_in(jax.random.key(0), 1)
    kr = jax.random.key_data(sample_key).astype(jnp.uint32).reshape(1, 2)
    tk = jnp.asarray(top_k, jnp.float32).reshape(1, 1)
    out = _build_call(R, V)(logits, tk, kr)
    return out.reshape(R)
